# baseline (speedup 1.0000x reference)
import functools

import jax
import jax.numpy as jnp
from jax import lax
from jax.experimental import pallas as pl
from jax.experimental.pallas import tpu as pltpu
from jax.experimental.pallas import tpu_sc as plsc

B = 1024
V = 100000
S = 128
NC, NS, L = 2, 16, 16
NW = NC * NS
RPW = B // NW


def _sc_gather(input_flat, target, samples):
    mesh = plsc.VectorSubcoreMesh(core_axis_name="c", subcore_axis_name="s")

    @functools.partial(
        pl.kernel,
        mesh=mesh,
        out_type=(
            jax.ShapeDtypeStruct((B,), jnp.float32),
            jax.ShapeDtypeStruct((B, S), jnp.float32),
        ),
        scratch_types=[
            pltpu.VMEM((RPW,), jnp.int32),
            pltpu.VMEM((RPW, S), jnp.int32),
            pltpu.VMEM((RPW,), jnp.float32),
            pltpu.VMEM((RPW, S), jnp.float32),
            pltpu.SemaphoreType.DMA,
        ],
    )
    def k(inp_hbm, tgt_hbm, smp_hbm, tout_hbm, sout_hbm,
          tflat_v, sflat_v, tsc_v, ssc_v, sem):
        wid = lax.axis_index("s") * NC + lax.axis_index("c")
        base = wid * RPW

        pltpu.sync_copy(tgt_hbm.at[pl.ds(base, RPW)], tflat_v)
        pltpu.sync_copy(smp_hbm.at[pl.ds(base, RPW)], sflat_v)

        for j in range(RPW // L):
            rows = base + j * L + lax.iota(jnp.int32, L)
            sl = pl.ds(j * L, L)
            tflat_v[sl] = tflat_v[sl] + rows * V
        copies = [pltpu.async_copy(inp_hbm.at[tflat_v], tsc_v, sem)]

        for r in range(RPW):
            row_off = (base + r) * V
            for j in range(S // L):
                sl = pl.ds(j * L, L)
                sflat_v[r, sl] = sflat_v[r, sl] + row_off
            copies.append(
                pltpu.async_copy(inp_hbm.at[sflat_v.at[r]], ssc_v.at[r], sem)
            )

        for c in copies:
            c.wait()

        pltpu.sync_copy(tsc_v, tout_hbm.at[pl.ds(base, RPW)])
        pltpu.sync_copy(ssc_v, sout_hbm.at[pl.ds(base, RPW)])

    return k(input_flat, target, samples)


def _tc_loss(tsc, ssc):
    def body(t_ref, s_ref, o_ref):
        x = t_ref[...] - s_ref[...]
        z = jnp.minimum(x, 0.0) - jnp.log1p(jnp.exp(-jnp.abs(x)))
        o_ref[0, 0] = -jnp.sum(z) * (1.0 / (B * S))

    return pl.pallas_call(
        body,
        out_shape=jax.ShapeDtypeStruct((1, 1), jnp.float32),
        out_specs=pl.BlockSpec(memory_space=pltpu.SMEM),
    )(tsc, ssc)


def kernel(input, target, samples):
    tsc, ssc = _sc_gather(
        input.reshape(-1), target.astype(jnp.int32), samples.astype(jnp.int32)
    )
    return _tc_loss(tsc.reshape(B, 1), ssc)[0, 0]


# speedup vs baseline: 13.4339x; 13.4339x over previous
import functools

import jax
import jax.numpy as jnp
from jax import lax
from jax.experimental import pallas as pl
from jax.experimental.pallas import tpu as pltpu
from jax.experimental.pallas import tpu_sc as plsc

B = 1024
V = 100000
S = 128
NC, NS, L = 2, 16, 16
NW = NC * NS
RPW = B // NW
LANES = 128


def _sc_gather(input_t, target, samples):
    mesh = plsc.VectorSubcoreMesh(core_axis_name="c", subcore_axis_name="s")

    @functools.partial(
        pl.kernel,
        mesh=mesh,
        out_type=(
            jax.ShapeDtypeStruct((B,), jnp.float32),
            jax.ShapeDtypeStruct((B, S), jnp.float32),
        ),
        scratch_types=[
            pltpu.VMEM((RPW * S,), jnp.int32),
            pltpu.VMEM((RPW,), jnp.int32),
            pltpu.VMEM((2, S, LANES), jnp.float32),
            pltpu.VMEM((RPW, LANES), jnp.float32),
            pltpu.VMEM((RPW, S), jnp.float32),
            pltpu.VMEM((RPW,), jnp.float32),
            pltpu.SemaphoreType.DMA,
        ],
        compiler_params=pltpu.CompilerParams(needs_layout_passes=False),
    )
    def k(inp_hbm, tgt_hbm, smp_hbm, tout_hbm, sout_hbm,
          idx_v, tid_v, lines_v, tlines_v, ssc_v, tsc_v, sem):
        wid = lax.axis_index("s") * NC + lax.axis_index("c")
        base = wid * RPW
        col0 = pl.multiple_of((wid // 4) * LANES, LANES)
        lane0 = base - col0

        def load_ids(r, _):
            pltpu.sync_copy(
                smp_hbm.at[base + r],
                idx_v.at[pl.ds(pl.multiple_of(S * r, S), S)],
            )
            return 0

        lax.fori_loop(0, RPW, load_ids, 0)
        pltpu.sync_copy(tgt_hbm.at[pl.ds(base, RPW)], tid_v)

        tcopy = pltpu.async_copy(
            inp_hbm.at[tid_v, pl.ds(col0, LANES)], tlines_v, sem
        )

        def chunk_copy(r):
            return pltpu.make_async_copy(
                inp_hbm.at[
                    idx_v.at[pl.ds(pl.multiple_of(S * r, S), S)],
                    pl.ds(col0, LANES),
                ],
                lines_v.at[r % 2], sem,
            )

        chunk_copy(0).start()
        chunk_copy(1).start()

        def step(r, _):
            chunk_copy(r).wait()

            @pl.when(r + 2 < RPW)
            def _():
                chunk_copy(r + 2).start()

            lvec = jnp.full((L,), lane0 + r, jnp.int32)
            buf = jnp.full((L,), r % 2, jnp.int32)
            for kk in range(S // L):
                rows = lax.iota(jnp.int32, L) + kk * L
                x = plsc.load_gather(lines_v, [buf, rows, lvec])
                ssc_v[r, pl.ds(kk * L, L)] = x
            return 0

        lax.fori_loop(0, RPW, step, 0)

        tcopy.wait()
        for kk in range(RPW // L):
            rows = lax.iota(jnp.int32, L) + kk * L
            x = plsc.load_gather(tlines_v, [rows, lane0 + rows])
            tsc_v[pl.ds(kk * L, L)] = x

        pltpu.sync_copy(ssc_v, sout_hbm.at[pl.ds(base, RPW)])
        pltpu.sync_copy(tsc_v, tout_hbm.at[pl.ds(base, RPW)])

    return k(input_t, target, samples)


def _tc_loss(tsc, ssc):
    def body(t_ref, s_ref, o_ref):
        x = t_ref[...] - s_ref[...]
        z = jnp.minimum(x, 0.0) - jnp.log1p(jnp.exp(-jnp.abs(x)))
        o_ref[0, 0] = -jnp.sum(z) * (1.0 / (B * S))

    return pl.pallas_call(
        body,
        out_shape=jax.ShapeDtypeStruct((1, 1), jnp.float32),
        out_specs=pl.BlockSpec(memory_space=pltpu.SMEM),
    )(tsc, ssc)


def kernel(input, target, samples):
    tsc, ssc = _sc_gather(
        input.T, target.astype(jnp.int32), samples.astype(jnp.int32)
    )
    return _tc_loss(tsc.reshape(B, 1), ssc)[0, 0]


# speedup vs baseline: 17.4278x; 1.2973x over previous
import functools

import jax
import jax.numpy as jnp
from jax import lax
from jax.experimental import pallas as pl
from jax.experimental.pallas import tpu as pltpu
from jax.experimental.pallas import tpu_sc as plsc

B = 1024
V = 100000
S = 128
NC, NS, L = 2, 16, 16
NW = NC * NS
RPW = B // NW
LANES = 128
NBUF = 4


def _sc_gather(input_t, target, samples):
    mesh = plsc.VectorSubcoreMesh(core_axis_name="c", subcore_axis_name="s")

    @functools.partial(
        pl.kernel,
        mesh=mesh,
        out_type=jax.ShapeDtypeStruct((B, S), jnp.float32),
        scratch_types=[
            pltpu.VMEM((RPW, S), jnp.int32),
            pltpu.VMEM((RPW,), jnp.int32),
            pltpu.VMEM((NBUF, S, LANES), jnp.float32),
            pltpu.VMEM((RPW, LANES), jnp.float32),
            pltpu.VMEM((RPW, S), jnp.float32),
            pltpu.VMEM((RPW,), jnp.float32),
            pltpu.SemaphoreType.DMA,
        ],
        compiler_params=pltpu.CompilerParams(needs_layout_passes=False),
    )
    def k(inp_hbm, tgt_hbm, smp_hbm, sout_hbm,
          idx_v, tid_v, lines_v, tlines_v, ssc_v, tsc_v, sem):
        wid = lax.axis_index("s") * NC + lax.axis_index("c")
        base = wid * RPW
        col0 = pl.multiple_of((wid // 4) * LANES, LANES)
        lane0 = base - col0

        pltpu.sync_copy(smp_hbm.at[pl.ds(base, RPW)], idx_v)
        pltpu.sync_copy(tgt_hbm.at[pl.ds(base, RPW)], tid_v)

        tcopy = pltpu.async_copy(
            inp_hbm.at[tid_v, pl.ds(col0, LANES)], tlines_v, sem
        )

        def chunk_copy(r):
            return pltpu.make_async_copy(
                inp_hbm.at[idx_v.at[r], pl.ds(col0, LANES)],
                lines_v.at[r % NBUF], sem,
            )

        for r in range(NBUF):
            chunk_copy(r).start()

        tcopy.wait()
        for kk in range(RPW // L):
            rows = lax.iota(jnp.int32, L) + kk * L
            x = plsc.load_gather(tlines_v, [rows, lane0 + rows])
            tsc_v[pl.ds(kk * L, L)] = x

        def step(r, _):
            chunk_copy(r).wait()

            @pl.when(r + NBUF < RPW)
            def _():
                chunk_copy(r + NBUF).start()

            tsp = plsc.load_gather(tsc_v, [jnp.full((L,), r, jnp.int32)])
            lvec = jnp.full((L,), lane0 + r, jnp.int32)
            buf = jnp.full((L,), r % NBUF, jnp.int32)
            for kk in range(S // L):
                rows = lax.iota(jnp.int32, L) + kk * L
                x = plsc.load_gather(lines_v, [buf, rows, lvec])
                ssc_v[r, pl.ds(kk * L, L)] = tsp - x
            return 0

        lax.fori_loop(0, RPW, step, 0)

        pltpu.sync_copy(ssc_v, sout_hbm.at[pl.ds(base, RPW)])

    return k(input_t, target, samples)


def _tc_loss(diff):
    def body(d_ref, o_ref):
        x = d_ref[...]
        z = jnp.minimum(x, 0.0) - jnp.log1p(jnp.exp(-jnp.abs(x)))
        o_ref[0, 0] = -jnp.sum(z) * (1.0 / (B * S))

    return pl.pallas_call(
        body,
        out_shape=jax.ShapeDtypeStruct((1, 1), jnp.float32),
        out_specs=pl.BlockSpec(memory_space=pltpu.SMEM),
    )(diff)


def kernel(input, target, samples):
    diff = _sc_gather(
        input.T, target.astype(jnp.int32), samples.astype(jnp.int32)
    )
    return _tc_loss(diff)[0, 0]
